# Initial kernel scaffold; baseline (speedup 1.0000x reference)
import jax
import jax.numpy as jnp
from jax.experimental import pallas as pl


def kernel(x, expert_indices, expert_weights, w1, w2, w3):
    raise NotImplementedError("write your pallas kernel here")


# speedup vs baseline: 6.7122x; 6.7122x over previous
import jax
import jax.numpy as jnp
from jax.experimental import pallas as pl

T = 32
DIM = 1024
INTER = 2048
E = 16
TOPK = 2
IB = 1024
NJ = INTER // IB


def _ffn_kernel(idx_ref, gw_ref, x_ref, w1_ref, w3_ref, w2_ref, out_ref):
    e = pl.program_id(0)
    j = pl.program_id(1)

    @pl.when((e == 0) & (j == 0))
    def _init():
        out_ref[...] = jnp.zeros_like(out_ref)

    x = x_ref[...]
    w1 = w1_ref[0]
    w3 = w3_ref[0]
    w2 = w2_ref[0]

    dn = (((1,), (1,)), ((), ()))
    x1 = jax.lax.dot_general(x, w1, dn, preferred_element_type=jnp.float32)
    x3 = jax.lax.dot_general(x, w3, dn, preferred_element_type=jnp.float32)
    h = x1 * jax.nn.sigmoid(x1) * x3
    part = jax.lax.dot_general(h, w2, dn, preferred_element_type=jnp.float32)

    idx = idx_ref[...]
    gw = gw_ref[...]
    scale = jnp.sum(jnp.where(idx == e, gw, 0.0), axis=1, keepdims=True)
    out_ref[...] += scale * part


def kernel(x, expert_indices, expert_weights, w1, w2, w3):
    idx = expert_indices.astype(jnp.int32)
    return pl.pallas_call(
        _ffn_kernel,
        grid=(E, NJ),
        in_specs=[
            pl.BlockSpec((T, TOPK), lambda e, j: (0, 0)),
            pl.BlockSpec((T, TOPK), lambda e, j: (0, 0)),
            pl.BlockSpec((T, DIM), lambda e, j: (0, 0)),
            pl.BlockSpec((1, IB, DIM), lambda e, j: (e, j, 0)),
            pl.BlockSpec((1, IB, DIM), lambda e, j: (e, j, 0)),
            pl.BlockSpec((1, DIM, IB), lambda e, j: (e, 0, j)),
        ],
        out_specs=pl.BlockSpec((T, DIM), lambda e, j: (0, 0)),
        out_shape=jax.ShapeDtypeStruct((T, DIM), jnp.float32),
    )(idx, expert_weights, x, w1, w3, w2)
